# baseline (speedup 1.0000x reference)
import functools

import jax
import jax.numpy as jnp
from jax import lax
from jax.experimental import pallas as pl
from jax.experimental.pallas import tpu as pltpu
from jax.experimental.pallas import tpu_sc as plsc

N = 10000
E = 320000
H = 128
R = 3
BS = 1024

NC = 2
NS = 16
LANES = 16

K = 128
NB = 160
EPT = NB * K
E_PAD = NS * EPT
SB = 5
SE = SB * K
NSUP = NB // SB
NPAIR = NSUP // 2

PH1 = 2
SPAN1 = N // (NC * PH1)
SL1 = R * SPAN1
PAD1 = 7680
RPT1 = PAD1 // NS
TRASH1 = PAD1
ACC1_ROWS = PAD1 + LANES

WIN = BS // NC
SL2 = R * WIN
PAD2 = 1536
RPT2 = PAD2 // NS
TRASH2 = PAD2
ACC2_ROWS = PAD2 + LANES

_mesh = lambda: plsc.VectorSubcoreMesh(
    core_axis_name="c", subcore_axis_name="s", num_cores=NC, num_subcores=NS)


def _stream_edges(h_hbm, key_hbm, src_hbm, keyb, srcb, idx2, row2,
                  gsem, ssem, csem, esem, acc_sh, cnt_sh, ones_v,
                  s, lo3, span3, trash):
    def cidx(eset, j, par):
        for g in range(K // LANES):
            kk = keyb[eset][pl.ds(j * K + g * LANES, LANES)] - lo3
            m = (kk >= 0) & (kk < span3)
            idx2[par][pl.ds(g * LANES, LANES)] = jnp.where(m, kk, trash)

    def fire_fetch(eset, sup):
        ebase = s * EPT + sup * SE
        pltpu.async_copy(key_hbm.at[pl.ds(ebase, SE)], keyb[eset], esem)
        pltpu.async_copy(src_hbm.at[pl.ds(ebase, SE)], srcb[eset], esem)

    def drain_fetch(eset, sup):
        ebase = s * EPT + sup * SE
        pltpu.make_async_copy(key_hbm.at[pl.ds(ebase, SE)], keyb[eset],
                              esem).wait()
        pltpu.make_async_copy(src_hbm.at[pl.ds(ebase, SE)], srcb[eset],
                              esem).wait()

    def fire_gather(eset, j, par):
        pltpu.async_copy(h_hbm.at[srcb[eset].at[pl.ds(j * K, K)]],
                         row2[par], gsem[par])

    def wait_gather(eset, j, par):
        pltpu.make_async_copy(h_hbm.at[srcb[eset].at[pl.ds(j * K, K)]],
                              row2[par], gsem[par]).wait()

    def fire_scatter(par):
        pltpu.async_copy(row2[par], acc_sh.at[idx2[par]], ssem[par], add=True)
        if cnt_sh is not None:
            pltpu.async_copy(ones_v, cnt_sh.at[idx2[par]], csem[par],
                             add=True)

    def wait_scatter(par):
        pltpu.make_async_copy(row2[par], acc_sh.at[idx2[par]],
                              ssem[par]).wait()
        if cnt_sh is not None:
            pltpu.make_async_copy(ones_v, cnt_sh.at[idx2[par]],
                                  csem[par]).wait()

    fire_fetch(0, 0)
    drain_fetch(0, 0)

    @pl.loop(0, NPAIR)
    def pair(i):
        for jj in range(2 * SB):
            par = jj % 2
            eset = 0 if jj < SB else 1
            j = jj if jj < SB else jj - SB

            if jj == 0:
                @pl.when(i > 0)
                def _():
                    drain_fetch(0, 2 * i)
                    wait_gather(1, SB - 1, 1)
                    fire_scatter(1)
                    wait_scatter(0)
                fire_fetch(1, 2 * i + 1)
            elif jj == 1:
                @pl.when(i > 0)
                def _():
                    wait_scatter(1)
            else:
                wait_scatter(par)

            if jj == SB:
                drain_fetch(1, 2 * i + 1)

            cidx(eset, j, par)
            fire_gather(eset, j, par)

            if jj > 0:
                peset = 0 if jj - 1 < SB else 1
                pj = jj - 1 if jj - 1 < SB else jj - 1 - SB
                wait_gather(peset, pj, 1 - par)
                fire_scatter(1 - par)

            if jj == SB:
                @pl.when(i < NPAIR - 1)
                def _():
                    fire_fetch(0, 2 * i + 2)

    wait_gather(1, SB - 1, 1)
    fire_scatter(1)
    wait_scatter(0)
    wait_scatter(1)


def _sc_l1(h, key, src, z, zc, e0):
    @functools.partial(
        pl.kernel,
        out_type=(
            jax.ShapeDtypeStruct((NC * PH1 * PAD1, H), jnp.float32),
            jax.ShapeDtypeStruct((NC * PH1 * PAD1, LANES), jnp.float32),
        ),
        mesh=_mesh(),
        scratch_types=[
            pltpu.VMEM_SHARED((ACC1_ROWS, H), jnp.float32),
            pltpu.VMEM_SHARED((ACC1_ROWS, LANES), jnp.float32),
            pltpu.VMEM((K, LANES), jnp.float32),
            pltpu.VMEM((SE,), jnp.int32),
            pltpu.VMEM((SE,), jnp.int32),
            pltpu.VMEM((SE,), jnp.int32),
            pltpu.VMEM((SE,), jnp.int32),
            pltpu.VMEM((K,), jnp.int32),
            pltpu.VMEM((K,), jnp.int32),
            pltpu.VMEM((K, H), jnp.float32),
            pltpu.VMEM((K, H), jnp.float32),
            pltpu.SemaphoreType.DMA,
            pltpu.SemaphoreType.DMA,
            pltpu.SemaphoreType.DMA,
            pltpu.SemaphoreType.DMA,
            pltpu.SemaphoreType.DMA,
            pltpu.SemaphoreType.DMA,
            pltpu.SemaphoreType.DMA,
        ],
    )
    def k(h_hbm, key_hbm, src_hbm, z_hbm, zc_hbm, e0_hbm,
          sums_hbm, cnt_hbm,
          acc_sh, cnt_sh, ones_v, key_a, key_b, src_a, src_b,
          idx_a, idx_b, row_a, row_b,
          gsem_a, gsem_b, ssem_a, ssem_b, csem_a, csem_b, esem):
        c = lax.axis_index("c")
        s = lax.axis_index("s")
        pltpu.sync_copy(e0_hbm, ones_v)
        for p in range(PH1):
            lo3 = ((c * PH1 + p) * SPAN1) * R
            pltpu.sync_copy(z_hbm.at[pl.ds(0, RPT1)],
                            acc_sh.at[pl.ds(s * RPT1, RPT1)])
            pltpu.sync_copy(zc_hbm.at[pl.ds(0, RPT1)],
                            cnt_sh.at[pl.ds(s * RPT1, RPT1)])

            @pl.when(s == 0)
            def _():
                pltpu.sync_copy(z_hbm.at[pl.ds(0, LANES)],
                                acc_sh.at[pl.ds(TRASH1, LANES)])
                pltpu.sync_copy(zc_hbm.at[pl.ds(0, LANES)],
                                cnt_sh.at[pl.ds(TRASH1, LANES)])

            plsc.subcore_barrier()
            _stream_edges(h_hbm, key_hbm, src_hbm,
                          (key_a, key_b), (src_a, src_b),
                          (idx_a, idx_b), (row_a, row_b),
                          (gsem_a, gsem_b), (ssem_a, ssem_b),
                          (csem_a, csem_b), esem,
                          acc_sh, cnt_sh, ones_v, s, lo3, SL1, TRASH1)
            plsc.subcore_barrier()
            ob = (c * PH1 + p) * PAD1 + s * RPT1
            pltpu.sync_copy(acc_sh.at[pl.ds(s * RPT1, RPT1)],
                            sums_hbm.at[pl.ds(ob, RPT1)])
            pltpu.sync_copy(cnt_sh.at[pl.ds(s * RPT1, RPT1)],
                            cnt_hbm.at[pl.ds(ob, RPT1)])

    return k(h, key, src, z, zc, e0)


def _sc_l2(h, keyw, src, z):
    @functools.partial(
        pl.kernel,
        out_type=jax.ShapeDtypeStruct((NC * PAD2, H), jnp.float32),
        mesh=_mesh(),
        scratch_types=[
            pltpu.VMEM_SHARED((ACC2_ROWS, H), jnp.float32),
            pltpu.VMEM((SE,), jnp.int32),
            pltpu.VMEM((SE,), jnp.int32),
            pltpu.VMEM((SE,), jnp.int32),
            pltpu.VMEM((SE,), jnp.int32),
            pltpu.VMEM((K,), jnp.int32),
            pltpu.VMEM((K,), jnp.int32),
            pltpu.VMEM((K, H), jnp.float32),
            pltpu.VMEM((K, H), jnp.float32),
            pltpu.SemaphoreType.DMA,
            pltpu.SemaphoreType.DMA,
            pltpu.SemaphoreType.DMA,
            pltpu.SemaphoreType.DMA,
            pltpu.SemaphoreType.DMA,
        ],
    )
    def k(h_hbm, key_hbm, src_hbm, z_hbm, out_hbm,
          acc_sh, key_a, key_b, src_a, src_b, idx_a, idx_b, row_a, row_b,
          gsem_a, gsem_b, ssem_a, ssem_b, esem):
        c = lax.axis_index("c")
        s = lax.axis_index("s")
        lo3 = (c * WIN) * R
        pltpu.sync_copy(z_hbm.at[pl.ds(0, RPT2)],
                        acc_sh.at[pl.ds(s * RPT2, RPT2)])

        @pl.when(s == 0)
        def _():
            pltpu.sync_copy(z_hbm.at[pl.ds(0, LANES)],
                            acc_sh.at[pl.ds(TRASH2, LANES)])

        plsc.subcore_barrier()
        _stream_edges(h_hbm, key_hbm, src_hbm,
                      (key_a, key_b), (src_a, src_b),
                      (idx_a, idx_b), (row_a, row_b),
                      (gsem_a, gsem_b), (ssem_a, ssem_b),
                      (None, None), esem,
                      acc_sh, None, None, s, lo3, SL2, TRASH2)
        plsc.subcore_barrier()
        pltpu.sync_copy(acc_sh.at[pl.ds(s * RPT2, RPT2)],
                        out_hbm.at[pl.ds(c * PAD2 + s * RPT2, RPT2)])

    return k(h, keyw, src, z)


def _tc_in_layer(x, W1, b1):
    BM = 2000

    def body(x_ref, w_ref, b_ref, o_ref):
        o_ref[...] = jax.nn.relu(
            jnp.dot(x_ref[...], w_ref[...],
                    preferred_element_type=jnp.float32) + b_ref[...])

    return pl.pallas_call(
        body,
        grid=(N // BM,),
        in_specs=[
            pl.BlockSpec((BM, H), lambda g: (g, 0)),
            pl.BlockSpec((H, H), lambda g: (0, 0)),
            pl.BlockSpec((1, H), lambda g: (0, 0)),
        ],
        out_specs=pl.BlockSpec((BM, H), lambda g: (g, 0)),
        out_shape=jax.ShapeDtypeStruct((N, H), jnp.float32),
    )(x, W1, b1)


def _tc_combine(h, root, bias, sums, rel, cnt_t):
    M = h.shape[0]
    BM = 2000 if M % 2000 == 0 else M
    G = M // BM

    def body(h_ref, root_ref, b_ref, s_ref, rel_ref, c_ref, o_ref):
        acc = jnp.dot(h_ref[...], root_ref[...],
                      preferred_element_type=jnp.float32) + b_ref[...]
        cnt = c_ref[0]
        for r in range(R):
            inv = 1.0 / jnp.maximum(cnt[r], 1.0)
            acc = acc + jnp.dot(s_ref[r] * inv[:, None], rel_ref[r],
                                preferred_element_type=jnp.float32)
        o_ref[...] = acc

    return pl.pallas_call(
        body,
        grid=(G,),
        in_specs=[
            pl.BlockSpec((BM, H), lambda g: (g, 0)),
            pl.BlockSpec((H, H), lambda g: (0, 0)),
            pl.BlockSpec((1, H), lambda g: (0, 0)),
            pl.BlockSpec((R, BM, H), lambda g: (0, g, 0)),
            pl.BlockSpec((R, H, H), lambda g: (0, 0, 0)),
            pl.BlockSpec((1, R, BM), lambda g: (g, 0, 0)),
        ],
        out_specs=pl.BlockSpec((BM, H), lambda g: (g, 0)),
        out_shape=jax.ShapeDtypeStruct((M, H), jnp.float32),
    )(h, root, bias, sums, rel, cnt_t)


def _tc_head(h2, W2, b2, Wc_p, bc_p):
    def body(h_ref, w2_ref, b2_ref, wc_ref, bc_ref, o_ref):
        t = jax.nn.relu(jnp.dot(h_ref[...], w2_ref[...],
                                preferred_element_type=jnp.float32) + b2_ref[...])
        o_ref[...] = jnp.dot(t, wc_ref[...],
                             preferred_element_type=jnp.float32) + bc_ref[...]

    return pl.pallas_call(
        body,
        grid=(1,),
        in_specs=[
            pl.BlockSpec((BS, H), lambda g: (0, 0)),
            pl.BlockSpec((H, H), lambda g: (0, 0)),
            pl.BlockSpec((1, H), lambda g: (0, 0)),
            pl.BlockSpec((H, H), lambda g: (0, 0)),
            pl.BlockSpec((1, H), lambda g: (0, 0)),
        ],
        out_specs=pl.BlockSpec((BS, H), lambda g: (0, 0)),
        out_shape=jax.ShapeDtypeStruct((BS, H), jnp.float32),
    )(h2, W2, b2, Wc_p, bc_p)


def kernel(x, edge_index, edge_type, batch_size, W1, b1, rel1, root1, bias1,
           rel2, root2, bias2, W2, b2, Wc, bc):
    src = edge_index[0]
    dst = edge_index[1]
    typ = edge_type.astype(jnp.int32)
    start = jnp.clip(jnp.asarray(batch_size, jnp.int32) - BS, 0, N - BS)
    extra = E_PAD - E
    key = jnp.pad(dst * R + typ, (0, extra), constant_values=(1 << 28))
    src = jnp.pad(src, (0, extra))

    z = jnp.zeros((RPT1, H), jnp.float32)
    zc = jnp.zeros((RPT1, LANES), jnp.float32)
    e0 = jnp.zeros((K, LANES), jnp.float32).at[:, 0].set(1.0)

    h0 = _tc_in_layer(x, W1, b1.reshape(1, H))

    sums1_raw, cnt_raw = _sc_l1(h0, key, src, z, zc, e0)
    sums1 = (sums1_raw.reshape(NC, PH1, PAD1, H)[:, :, :SL1]
             .reshape(NC, PH1, SPAN1, R, H)
             .transpose(3, 0, 1, 2, 4).reshape(R, N, H))
    cnt = (cnt_raw.reshape(NC, PH1, PAD1, LANES)[:, :, :SL1, 0]
           .reshape(NC, PH1, SPAN1, R)
           .transpose(3, 0, 1, 2).reshape(R, N))

    cnt1_t = cnt.reshape(R, N // 2000, 2000).transpose(1, 0, 2)
    h1 = _tc_combine(h0, root1, bias1.reshape(1, H), sums1, rel1, cnt1_t)

    keyw = key - 3 * start
    sums2 = (_sc_l2(h1, keyw, src, z)
             .reshape(NC, PAD2, H)[:, :SL2]
             .reshape(NC, WIN, R, H)
             .transpose(2, 0, 1, 3).reshape(R, BS, H))

    h1w = lax.dynamic_slice(h1, (start, 0), (BS, H))
    cnt2 = lax.dynamic_slice(cnt, (0, start), (R, BS)).reshape(1, R, BS)
    h2 = _tc_combine(h1w, root2, bias2.reshape(1, H), sums2, rel2, cnt2)

    Wc_p = jnp.zeros((H, H), jnp.float32).at[:, :2].set(Wc)
    bc_p = jnp.zeros((1, H), jnp.float32).at[0, :2].set(bc)
    logits = _tc_head(h2, W2, b2.reshape(1, H), Wc_p, bc_p)
    return logits[:, :2]


# speedup vs baseline: 1.9277x; 1.9277x over previous
import functools

import jax
import jax.numpy as jnp
from jax import lax
from jax.experimental import pallas as pl
from jax.experimental.pallas import tpu as pltpu
from jax.experimental.pallas import tpu_sc as plsc

N = 10000
E = 320000
H = 128
R = 3
BS = 1024

NC = 2
NS = 16
LANES = 16

K = 80
NB = 250
EPT = NB * K
SB = 5
SE = SB * K
NSUP = NB // SB
NPAIR = NSUP // 2

PH1 = 2
SPAN1 = N // (NC * PH1)
SL1 = R * SPAN1
PAD1 = 7680
RPT1 = PAD1 // NS
TRASH1 = PAD1
ACC1_ROWS = PAD1 + LANES

WIN = BS // NC
SL2 = R * WIN
PAD2 = 1536
RPT2 = PAD2 // NS
TRASH2 = PAD2
ACC2_ROWS = PAD2 + LANES

_mesh = lambda: plsc.VectorSubcoreMesh(
    core_axis_name="c", subcore_axis_name="s", num_cores=NC, num_subcores=NS)


def _stream_edges(h_hbm, key_hbm, src_hbm, keyb, srcb, idx2, row2,
                  gsem, ssem, csem, esem, acc_sh, cnt_sh, ones_v,
                  s, lo3, span3, trash):
    def cidx(eset, j, par):
        for g in range(K // LANES):
            kk = keyb[eset][pl.ds(j * K + g * LANES, LANES)] - lo3
            m = (kk >= 0) & (kk < span3)
            idx2[par][pl.ds(g * LANES, LANES)] = jnp.where(m, kk, trash)

    def fire_fetch(eset, sup):
        ebase = s * EPT + sup * SE
        pltpu.async_copy(key_hbm.at[pl.ds(ebase, SE)], keyb[eset], esem)
        pltpu.async_copy(src_hbm.at[pl.ds(ebase, SE)], srcb[eset], esem)

    def drain_fetch(eset, sup):
        ebase = s * EPT + sup * SE
        pltpu.make_async_copy(key_hbm.at[pl.ds(ebase, SE)], keyb[eset],
                              esem).wait()
        pltpu.make_async_copy(src_hbm.at[pl.ds(ebase, SE)], srcb[eset],
                              esem).wait()

    def fire_gather(eset, j, par):
        pltpu.async_copy(h_hbm.at[srcb[eset].at[pl.ds(j * K, K)]],
                         row2[par], gsem[par])

    def wait_gather(eset, j, par):
        pltpu.make_async_copy(h_hbm.at[srcb[eset].at[pl.ds(j * K, K)]],
                              row2[par], gsem[par]).wait()

    def fire_scatter(par):
        pltpu.async_copy(row2[par], acc_sh.at[idx2[par]], ssem[par], add=True)
        if cnt_sh is not None:
            pltpu.async_copy(ones_v, cnt_sh.at[idx2[par]], csem[par],
                             add=True)

    def wait_scatter(par):
        pltpu.make_async_copy(row2[par], acc_sh.at[idx2[par]],
                              ssem[par]).wait()
        if cnt_sh is not None:
            pltpu.make_async_copy(ones_v, cnt_sh.at[idx2[par]],
                                  csem[par]).wait()

    fire_fetch(0, 0)
    drain_fetch(0, 0)

    @pl.loop(0, NPAIR)
    def pair(i):
        for jj in range(2 * SB):
            par = jj % 2
            eset = 0 if jj < SB else 1
            j = jj if jj < SB else jj - SB

            if jj == 0:
                @pl.when(i > 0)
                def _():
                    drain_fetch(0, 2 * i)
                    wait_gather(1, SB - 1, 1)
                    fire_scatter(1)
                    wait_scatter(0)
                fire_fetch(1, 2 * i + 1)
            elif jj == 1:
                @pl.when(i > 0)
                def _():
                    wait_scatter(1)
            else:
                wait_scatter(par)

            if jj == SB:
                drain_fetch(1, 2 * i + 1)

            cidx(eset, j, par)
            fire_gather(eset, j, par)

            if jj > 0:
                peset = 0 if jj - 1 < SB else 1
                pj = jj - 1 if jj - 1 < SB else jj - 1 - SB
                wait_gather(peset, pj, 1 - par)
                fire_scatter(1 - par)

            if jj == SB:
                @pl.when(i < NPAIR - 1)
                def _():
                    fire_fetch(0, 2 * i + 2)

    wait_gather(1, SB - 1, 1)
    fire_scatter(1)
    wait_scatter(0)
    wait_scatter(1)


def _sc_l1(h, key, src, z, zc, e0):
    @functools.partial(
        pl.kernel,
        out_type=(
            jax.ShapeDtypeStruct((NC * PH1 * PAD1, H), jnp.float32),
            jax.ShapeDtypeStruct((NC * PH1 * PAD1, LANES), jnp.float32),
        ),
        mesh=_mesh(),
        scratch_types=[
            pltpu.VMEM_SHARED((ACC1_ROWS, H), jnp.float32),
            pltpu.VMEM_SHARED((ACC1_ROWS, LANES), jnp.float32),
            pltpu.VMEM((K, LANES), jnp.float32),
            pltpu.VMEM((SE,), jnp.int32),
            pltpu.VMEM((SE,), jnp.int32),
            pltpu.VMEM((SE,), jnp.int32),
            pltpu.VMEM((SE,), jnp.int32),
            pltpu.VMEM((K,), jnp.int32),
            pltpu.VMEM((K,), jnp.int32),
            pltpu.VMEM((K, H), jnp.float32),
            pltpu.VMEM((K, H), jnp.float32),
            pltpu.SemaphoreType.DMA,
            pltpu.SemaphoreType.DMA,
            pltpu.SemaphoreType.DMA,
            pltpu.SemaphoreType.DMA,
            pltpu.SemaphoreType.DMA,
            pltpu.SemaphoreType.DMA,
            pltpu.SemaphoreType.DMA,
        ],
    )
    def k(h_hbm, key_hbm, src_hbm, z_hbm, zc_hbm, e0_hbm,
          sums_hbm, cnt_hbm,
          acc_sh, cnt_sh, ones_v, key_a, key_b, src_a, src_b,
          idx_a, idx_b, row_a, row_b,
          gsem_a, gsem_b, ssem_a, ssem_b, csem_a, csem_b, esem):
        c = lax.axis_index("c")
        s = lax.axis_index("s")
        pltpu.sync_copy(e0_hbm, ones_v)
        for p in range(PH1):
            lo3 = ((c * PH1 + p) * SPAN1) * R
            pltpu.sync_copy(z_hbm.at[pl.ds(0, RPT1)],
                            acc_sh.at[pl.ds(s * RPT1, RPT1)])
            pltpu.sync_copy(zc_hbm.at[pl.ds(0, RPT1)],
                            cnt_sh.at[pl.ds(s * RPT1, RPT1)])

            @pl.when(s == 0)
            def _():
                pltpu.sync_copy(z_hbm.at[pl.ds(0, LANES)],
                                acc_sh.at[pl.ds(TRASH1, LANES)])
                pltpu.sync_copy(zc_hbm.at[pl.ds(0, LANES)],
                                cnt_sh.at[pl.ds(TRASH1, LANES)])

            plsc.subcore_barrier()
            _stream_edges(h_hbm, key_hbm, src_hbm,
                          (key_a, key_b), (src_a, src_b),
                          (idx_a, idx_b), (row_a, row_b),
                          (gsem_a, gsem_b), (ssem_a, ssem_b),
                          (csem_a, csem_b), esem,
                          acc_sh, cnt_sh, ones_v, s, lo3, SL1, TRASH1)
            plsc.subcore_barrier()
            ob = (c * PH1 + p) * PAD1 + s * RPT1
            pltpu.sync_copy(acc_sh.at[pl.ds(s * RPT1, RPT1)],
                            sums_hbm.at[pl.ds(ob, RPT1)])
            pltpu.sync_copy(cnt_sh.at[pl.ds(s * RPT1, RPT1)],
                            cnt_hbm.at[pl.ds(ob, RPT1)])

    return k(h, key, src, z, zc, e0)


def _sc_l2(h, keyw, src, z):
    @functools.partial(
        pl.kernel,
        out_type=jax.ShapeDtypeStruct((NC * PAD2, H), jnp.float32),
        mesh=_mesh(),
        scratch_types=[
            pltpu.VMEM_SHARED((ACC2_ROWS, H), jnp.float32),
            pltpu.VMEM((SE,), jnp.int32),
            pltpu.VMEM((SE,), jnp.int32),
            pltpu.VMEM((SE,), jnp.int32),
            pltpu.VMEM((SE,), jnp.int32),
            pltpu.VMEM((K,), jnp.int32),
            pltpu.VMEM((K,), jnp.int32),
            pltpu.VMEM((K, H), jnp.float32),
            pltpu.VMEM((K, H), jnp.float32),
            pltpu.SemaphoreType.DMA,
            pltpu.SemaphoreType.DMA,
            pltpu.SemaphoreType.DMA,
            pltpu.SemaphoreType.DMA,
            pltpu.SemaphoreType.DMA,
        ],
    )
    def k(h_hbm, key_hbm, src_hbm, z_hbm, out_hbm,
          acc_sh, key_a, key_b, src_a, src_b, idx_a, idx_b, row_a, row_b,
          gsem_a, gsem_b, ssem_a, ssem_b, esem):
        c = lax.axis_index("c")
        s = lax.axis_index("s")
        lo3 = (c * WIN) * R
        pltpu.sync_copy(z_hbm.at[pl.ds(0, RPT2)],
                        acc_sh.at[pl.ds(s * RPT2, RPT2)])

        @pl.when(s == 0)
        def _():
            pltpu.sync_copy(z_hbm.at[pl.ds(0, LANES)],
                            acc_sh.at[pl.ds(TRASH2, LANES)])

        plsc.subcore_barrier()
        _stream_edges(h_hbm, key_hbm, src_hbm,
                      (key_a, key_b), (src_a, src_b),
                      (idx_a, idx_b), (row_a, row_b),
                      (gsem_a, gsem_b), (ssem_a, ssem_b),
                      (None, None), esem,
                      acc_sh, None, None, s, lo3, SL2, TRASH2)
        plsc.subcore_barrier()
        pltpu.sync_copy(acc_sh.at[pl.ds(s * RPT2, RPT2)],
                        out_hbm.at[pl.ds(c * PAD2 + s * RPT2, RPT2)])

    return k(h, keyw, src, z)


def _tc_in_layer(x, W1, b1):
    BM = 2000

    def body(x_ref, w_ref, b_ref, o_ref):
        o_ref[...] = jax.nn.relu(
            jnp.dot(x_ref[...], w_ref[...],
                    preferred_element_type=jnp.float32) + b_ref[...])

    return pl.pallas_call(
        body,
        grid=(N // BM,),
        in_specs=[
            pl.BlockSpec((BM, H), lambda g: (g, 0)),
            pl.BlockSpec((H, H), lambda g: (0, 0)),
            pl.BlockSpec((1, H), lambda g: (0, 0)),
        ],
        out_specs=pl.BlockSpec((BM, H), lambda g: (g, 0)),
        out_shape=jax.ShapeDtypeStruct((N, H), jnp.float32),
    )(x, W1, b1)


def _tc_combine(h, root, bias, sums, rel, cnt_t):
    M = h.shape[0]
    BM = 2000 if M % 2000 == 0 else M
    G = M // BM

    def body(h_ref, root_ref, b_ref, s_ref, rel_ref, c_ref, o_ref):
        acc = jnp.dot(h_ref[...], root_ref[...],
                      preferred_element_type=jnp.float32) + b_ref[...]
        cnt = c_ref[0]
        for r in range(R):
            inv = 1.0 / jnp.maximum(cnt[r], 1.0)
            acc = acc + jnp.dot(s_ref[r] * inv[:, None], rel_ref[r],
                                preferred_element_type=jnp.float32)
        o_ref[...] = acc

    return pl.pallas_call(
        body,
        grid=(G,),
        in_specs=[
            pl.BlockSpec((BM, H), lambda g: (g, 0)),
            pl.BlockSpec((H, H), lambda g: (0, 0)),
            pl.BlockSpec((1, H), lambda g: (0, 0)),
            pl.BlockSpec((R, BM, H), lambda g: (0, g, 0)),
            pl.BlockSpec((R, H, H), lambda g: (0, 0, 0)),
            pl.BlockSpec((1, R, BM), lambda g: (g, 0, 0)),
        ],
        out_specs=pl.BlockSpec((BM, H), lambda g: (g, 0)),
        out_shape=jax.ShapeDtypeStruct((M, H), jnp.float32),
    )(h, root, bias, sums, rel, cnt_t)


def _tc_head(h2, W2, b2, Wc_p, bc_p):
    def body(h_ref, w2_ref, b2_ref, wc_ref, bc_ref, o_ref):
        t = jax.nn.relu(jnp.dot(h_ref[...], w2_ref[...],
                                preferred_element_type=jnp.float32) + b2_ref[...])
        o_ref[...] = jnp.dot(t, wc_ref[...],
                             preferred_element_type=jnp.float32) + bc_ref[...]

    return pl.pallas_call(
        body,
        grid=(1,),
        in_specs=[
            pl.BlockSpec((BS, H), lambda g: (0, 0)),
            pl.BlockSpec((H, H), lambda g: (0, 0)),
            pl.BlockSpec((1, H), lambda g: (0, 0)),
            pl.BlockSpec((H, H), lambda g: (0, 0)),
            pl.BlockSpec((1, H), lambda g: (0, 0)),
        ],
        out_specs=pl.BlockSpec((BS, H), lambda g: (0, 0)),
        out_shape=jax.ShapeDtypeStruct((BS, H), jnp.float32),
    )(h2, W2, b2, Wc_p, bc_p)


def kernel(x, edge_index, edge_type, batch_size, W1, b1, rel1, root1, bias1,
           rel2, root2, bias2, W2, b2, Wc, bc):
    src = edge_index[0]
    dst = edge_index[1]
    typ = edge_type.astype(jnp.int32)
    start = jnp.clip(jnp.asarray(batch_size, jnp.int32) - BS, 0, N - BS)
    key = dst * R + typ

    z = jnp.zeros((RPT1, H), jnp.float32)
    zc = jnp.zeros((RPT1, LANES), jnp.float32)
    e0 = jnp.zeros((K, LANES), jnp.float32).at[:, 0].set(1.0)

    h0 = _tc_in_layer(x, W1, b1.reshape(1, H))

    sums1_raw, cnt_raw = _sc_l1(h0, key, src, z, zc, e0)
    sums1 = (sums1_raw.reshape(NC, PH1, PAD1, H)[:, :, :SL1]
             .reshape(NC, PH1, SPAN1, R, H)
             .transpose(3, 0, 1, 2, 4).reshape(R, N, H))
    cnt = (cnt_raw.reshape(NC, PH1, PAD1, LANES)[:, :, :SL1, 0]
           .reshape(NC, PH1, SPAN1, R)
           .transpose(3, 0, 1, 2).reshape(R, N))

    cnt1_t = cnt.reshape(R, N // 2000, 2000).transpose(1, 0, 2)
    h1 = _tc_combine(h0, root1, bias1.reshape(1, H), sums1, rel1, cnt1_t)

    keyw = key - 3 * start
    sums2 = (_sc_l2(h1, keyw, src, z)
             .reshape(NC, PAD2, H)[:, :SL2]
             .reshape(NC, WIN, R, H)
             .transpose(2, 0, 1, 3).reshape(R, BS, H))

    h1w = lax.dynamic_slice(h1, (start, 0), (BS, H))
    cnt2 = lax.dynamic_slice(cnt, (0, start), (R, BS)).reshape(1, R, BS)
    h2 = _tc_combine(h1w, root2, bias2.reshape(1, H), sums2, rel2, cnt2)

    Wc_p = jnp.zeros((H, H), jnp.float32).at[:, :2].set(Wc)
    bc_p = jnp.zeros((1, H), jnp.float32).at[0, :2].set(bc)
    logits = _tc_head(h2, W2, b2.reshape(1, H), Wc_p, bc_p)
    return logits[:, :2]
